# baseline (speedup 1.0000x reference)
import functools

import jax
import jax.numpy as jnp
from jax import lax
from jax.experimental import pallas as pl
from jax.experimental.pallas import tpu as pltpu
from jax.experimental.pallas import tpu_sc as plsc

NUM_ROWS = 1000000
DIM = 32
BATCH = 16384
HIST = 50

_info = plsc.get_sparse_core_info()
NC = _info.num_cores
NS = _info.num_subcores
NW = NC * NS

BLK = BATCH // NW
NBT = BLK // 128
NDT = DIM // 8

CH = 512
NFULL = NUM_ROWS // CH
TAIL = NUM_ROWS - NFULL * CH
NKP = (-(-NFULL // NW) + 1) // 2

_mesh = plsc.VectorSubcoreMesh(core_axis_name="c", subcore_axis_name="s")


def _make_repack():
    @functools.partial(
        pl.kernel,
        mesh=_mesh,
        out_type=jax.ShapeDtypeStruct((NUM_ROWS * DIM,), jnp.float32),
        scratch_types=[
            pltpu.VMEM((32, CH + 1), jnp.float32),
            pltpu.VMEM((32, CH + 1), jnp.float32),
            pltpu.VMEM((CH * DIM,), jnp.float32),
            pltpu.VMEM((CH * DIM,), jnp.float32),
            pltpu.SemaphoreType.DMA,
            pltpu.SemaphoreType.DMA,
            pltpu.SemaphoreType.DMA,
            pltpu.SemaphoreType.DMA,
        ],
        compiler_params=pltpu.CompilerParams(
            use_tc_tiling_on_sc=True, needs_layout_passes=False
        ),
    )
    def repack_kernel(wt_hbm, tail_hbm, wlin, Sa, Sb, Ta, Tb,
                      is0, is1, os0, os1):
        wid = lax.axis_index("s") * NC + lax.axis_index("c")

        iota16 = lax.iota(jnp.int32, 16)

        def chunk_of(k):
            return wid + k * NW

        def fire_in(k, S, isem):
            @pl.when(chunk_of(k) < NFULL)
            def _():
                pltpu.async_copy(
                    wt_hbm.at[:, pl.ds(chunk_of(k) * CH, CH)],
                    S.at[:, pl.ds(0, CH)],
                    isem,
                )

        def drain_in(S, isem):
            pltpu.make_async_copy(
                wt_hbm.at[:, pl.ds(0, CH)], S.at[:, pl.ds(0, CH)], isem
            ).wait()

        def drain_out(T, osem):
            pltpu.make_async_copy(T, wlin.at[pl.ds(0, CH * DIM)], osem).wait()

        def work(k, S, T, isem, osem, t):
            @pl.when(chunk_of(k) < NFULL)
            def _():
                drain_in(S, isem)

                @pl.when(t > 0)
                def _():
                    drain_out(T, osem)

                @plsc.parallel_loop(0, 2 * CH, 1, unroll=8)
                def _body(j):
                    bc = lax.shift_right_logical(j, 1)
                    d0 = lax.bitwise_and(j, 1) * 16
                    rows = iota16 + d0
                    cols = jnp.full((16,), bc, jnp.int32)
                    v = plsc.load_gather(S, [rows, cols])
                    T[pl.ds(bc * 32 + d0, 16)] = v

                pltpu.async_copy(
                    T, wlin.at[pl.ds(chunk_of(k) * CH * DIM, CH * DIM)], osem
                )
                fire_in(k + 2, S, isem)

        fire_in(0, Sa, is0)
        fire_in(1, Sb, is1)

        def pair(t, carry):
            work(2 * t, Sa, Ta, is0, os0, t)
            work(2 * t + 1, Sb, Tb, is1, os1, t)
            return carry

        lax.fori_loop(0, NKP, pair, 0)

        drain_out(Ta, os0)
        drain_out(Tb, os1)

        @pl.when(wid == 0)
        def _():
            pltpu.sync_copy(tail_hbm, Ta.at[pl.ds(0, TAIL * DIM)])
            pltpu.sync_copy(
                Ta.at[pl.ds(0, TAIL * DIM)],
                wlin.at[pl.ds(NFULL * CH * DIM, TAIL * DIM)],
            )

    return repack_kernel


def _make_gather():
    @functools.partial(
        pl.kernel,
        mesh=_mesh,
        out_type=jax.ShapeDtypeStruct((HIST, NDT, BATCH // 128, 8, 128),
                                      jnp.float32),
        scratch_types=[
            pltpu.VMEM((HIST, NBT, 128), jnp.int32),
            pltpu.VMEM((2, BLK, DIM), jnp.float32),
            pltpu.VMEM((2, NBT, NDT, 8, 131), jnp.float32),
            pltpu.SemaphoreType.DMA,
            pltpu.SemaphoreType.DMA,
            pltpu.SemaphoreType.DMA,
            pltpu.SemaphoreType.DMA,
        ],
        compiler_params=pltpu.CompilerParams(
            use_tc_tiling_on_sc=False, needs_layout_passes=False
        ),
    )
    def gather_kernel(x_hbm, w_hbm, o5, idx_v, G, T, gs0, gs1, os0, os1):
        wid = lax.axis_index("s") * NC + lax.axis_index("c")
        pltpu.sync_copy(x_hbm.at[:, pl.ds(wid * NBT, NBT), :], idx_v)

        def fire(h, buf, gsem):
            for q in range(NBT):
                pltpu.async_copy(
                    w_hbm.at[idx_v.at[h, q]],
                    G.at[buf, pl.ds(q * 128, 128)],
                    gsem,
                )

        def drain_g(buf, gsem):
            pltpu.make_async_copy(
                w_hbm.at[pl.ds(0, BLK)], G.at[buf], gsem
            ).wait()

        def drain_o(buf, osem):
            pltpu.make_async_copy(
                T.at[buf, :, :, :, pl.ds(0, 128)],
                o5.at[0, :, pl.ds(wid * NBT, NBT), :, :],
                osem,
            ).wait()

        iota16 = lax.iota(jnp.int32, 16)
        dt_lo = lax.shift_right_logical(iota16, 3)
        dt_hi = dt_lo + 2
        dr_v = lax.bitwise_and(iota16, 7)

        def transpose(buf):
            @plsc.parallel_loop(0, BLK, 1, unroll=8)
            def _body(r):
                q = lax.shift_right_logical(r, 7)
                bc = lax.bitwise_and(r, 127)
                qv = jnp.full((16,), q, jnp.int32)
                bcv = jnp.full((16,), bc, jnp.int32)
                v0 = G[buf, r, pl.ds(0, 16)]
                v1 = G[buf, r, pl.ds(16, 16)]
                plsc.store_scatter(T.at[buf], [qv, dt_lo, dr_v, bcv], v0)
                plsc.store_scatter(T.at[buf], [qv, dt_hi, dr_v, bcv], v1)

        def step(h, buf, gsem, osem, t):
            drain_g(buf, gsem)

            @pl.when(t > 0)
            def _():
                drain_o(buf, osem)

            transpose(buf)
            for q in range(NBT):
                pltpu.async_copy(
                    T.at[buf, q, :, :, pl.ds(0, 128)],
                    o5.at[h, :, wid * NBT + q, :, :],
                    osem,
                )

            @pl.when(h + 2 < HIST)
            def _():
                fire(h + 2, buf, gsem)

        fire(0, 0, gs0)
        fire(1, 1, gs1)

        def pair(t, carry):
            step(2 * t, 0, gs0, os0, t)
            step(2 * t + 1, 1, gs1, os1, t)
            return carry

        lax.fori_loop(0, HIST // 2, pair, 0)
        drain_o(0, os0)
        drain_o(1, os1)

    return gather_kernel


_repack = _make_repack()
_gather = _make_gather()


@jax.jit
def kernel(x, weight):
    xv = x.T.reshape(HIST, BATCH // 128, 128).astype(jnp.int32)
    tail = lax.slice(weight, (NFULL * CH, 0), (NUM_ROWS, DIM)).reshape(-1)
    wlin = _repack(weight.T, tail)
    o5 = _gather(xv, wlin.reshape(NUM_ROWS, DIM))
    out = (
        o5.transpose(0, 1, 3, 2, 4)
        .reshape(HIST, DIM, BATCH)
        .transpose(2, 0, 1)
    )
    return out


# speedup vs baseline: 1.0625x; 1.0625x over previous
import functools

import jax
import jax.numpy as jnp
from jax import lax
from jax.experimental import pallas as pl
from jax.experimental.pallas import tpu as pltpu
from jax.experimental.pallas import tpu_sc as plsc

NUM_ROWS = 1000000
DIM = 32
BATCH = 16384
HIST = 50

_info = plsc.get_sparse_core_info()
NC = _info.num_cores
NS = _info.num_subcores
NW = NC * NS

BLK = BATCH // NW
NBT = BLK // 128
NDT = DIM // 8

CH = 128
NFULL = NUM_ROWS // CH
TAIL = NUM_ROWS - NFULL * CH
NKP = (-(-NFULL // NW) + 1) // 2

_mesh = plsc.VectorSubcoreMesh(core_axis_name="c", subcore_axis_name="s")


def _make_repack():
    @functools.partial(
        pl.kernel,
        mesh=_mesh,
        out_type=jax.ShapeDtypeStruct((NUM_ROWS * DIM,), jnp.float32),
        scratch_types=[
            pltpu.VMEM((32, CH + 8), jnp.float32),
            pltpu.VMEM((32, CH + 8), jnp.float32),
            pltpu.VMEM((CH * DIM,), jnp.float32),
            pltpu.VMEM((CH * DIM,), jnp.float32),
            pltpu.SemaphoreType.DMA,
            pltpu.SemaphoreType.DMA,
            pltpu.SemaphoreType.DMA,
            pltpu.SemaphoreType.DMA,
        ],
        compiler_params=pltpu.CompilerParams(
            use_tc_tiling_on_sc=True, needs_layout_passes=False
        ),
    )
    def repack_kernel(wt_hbm, tail_hbm, wlin, Sa, Sb, Ta, Tb,
                      is0, is1, os0, os1):
        wid = lax.axis_index("s") * NC + lax.axis_index("c")

        iota16 = lax.iota(jnp.int32, 16)

        def chunk_of(k):
            return wid + k * NW

        def fire_in(k, S, isem):
            @pl.when(chunk_of(k) < NFULL)
            def _():
                pltpu.async_copy(
                    wt_hbm.at[:, pl.ds(chunk_of(k) * CH, CH)],
                    S.at[:, pl.ds(0, CH)],
                    isem,
                )

        def drain_in(S, isem):
            pltpu.make_async_copy(
                wt_hbm.at[:, pl.ds(0, CH)], S.at[:, pl.ds(0, CH)], isem
            ).wait()

        def drain_out(T, osem):
            pltpu.make_async_copy(T, wlin.at[pl.ds(0, CH * DIM)], osem).wait()

        def work(k, S, T, isem, osem, t):
            @pl.when(chunk_of(k) < NFULL)
            def _():
                drain_in(S, isem)

                @pl.when(t > 0)
                def _():
                    drain_out(T, osem)

                @plsc.parallel_loop(0, 2 * CH, 1, unroll=8)
                def _body(j):
                    bc = lax.shift_right_logical(j, 1)
                    d0 = lax.bitwise_and(j, 1) * 16
                    rows = iota16 + d0
                    cols = jnp.full((16,), bc, jnp.int32)
                    v = plsc.load_gather(S, [rows, cols])
                    T[pl.ds(bc * 32 + d0, 16)] = v

                pltpu.async_copy(
                    T, wlin.at[pl.ds(chunk_of(k) * CH * DIM, CH * DIM)], osem
                )
                fire_in(k + 2, S, isem)

        fire_in(0, Sa, is0)
        fire_in(1, Sb, is1)

        def pair(t, carry):
            work(2 * t, Sa, Ta, is0, os0, t)
            work(2 * t + 1, Sb, Tb, is1, os1, t)
            return carry

        lax.fori_loop(0, NKP, pair, 0)

        drain_out(Ta, os0)
        drain_out(Tb, os1)

        @pl.when(wid == 0)
        def _():
            pltpu.sync_copy(tail_hbm, Ta.at[pl.ds(0, TAIL * DIM)])
            pltpu.sync_copy(
                Ta.at[pl.ds(0, TAIL * DIM)],
                wlin.at[pl.ds(NFULL * CH * DIM, TAIL * DIM)],
            )

    return repack_kernel


def _make_gather():
    @functools.partial(
        pl.kernel,
        mesh=_mesh,
        out_type=jax.ShapeDtypeStruct((HIST, NDT, BATCH // 128, 8, 128),
                                      jnp.float32),
        scratch_types=[
            pltpu.VMEM((HIST, NBT, 128), jnp.int32),
            pltpu.VMEM((2, BLK, DIM), jnp.float32),
            pltpu.VMEM((2, NBT, NDT, 8, 131), jnp.float32),
            pltpu.SemaphoreType.DMA,
            pltpu.SemaphoreType.DMA,
            pltpu.SemaphoreType.DMA,
            pltpu.SemaphoreType.DMA,
        ],
        compiler_params=pltpu.CompilerParams(
            use_tc_tiling_on_sc=False, needs_layout_passes=False
        ),
    )
    def gather_kernel(x_hbm, w_hbm, o5, idx_v, G, T, gs0, gs1, os0, os1):
        wid = lax.axis_index("s") * NC + lax.axis_index("c")
        pltpu.sync_copy(x_hbm.at[:, pl.ds(wid * NBT, NBT), :], idx_v)

        def fire(h, buf, gsem):
            for q in range(NBT):
                pltpu.async_copy(
                    w_hbm.at[idx_v.at[h, q]],
                    G.at[buf, pl.ds(q * 128, 128)],
                    gsem,
                )

        def drain_g(buf, gsem):
            pltpu.make_async_copy(
                w_hbm.at[pl.ds(0, BLK)], G.at[buf], gsem
            ).wait()

        def drain_o(buf, osem):
            pltpu.make_async_copy(
                T.at[buf, :, :, :, pl.ds(0, 128)],
                o5.at[0, :, pl.ds(wid * NBT, NBT), :, :],
                osem,
            ).wait()

        iota16 = lax.iota(jnp.int32, 16)
        dt_lo = lax.shift_right_logical(iota16, 3)
        dt_hi = dt_lo + 2
        dr_v = lax.bitwise_and(iota16, 7)

        def transpose(buf):
            @plsc.parallel_loop(0, BLK, 1, unroll=8)
            def _body(r):
                q = lax.shift_right_logical(r, 7)
                bc = lax.bitwise_and(r, 127)
                qv = jnp.full((16,), q, jnp.int32)
                bcv = jnp.full((16,), bc, jnp.int32)
                v0 = G[buf, r, pl.ds(0, 16)]
                v1 = G[buf, r, pl.ds(16, 16)]
                plsc.store_scatter(T.at[buf], [qv, dt_lo, dr_v, bcv], v0)
                plsc.store_scatter(T.at[buf], [qv, dt_hi, dr_v, bcv], v1)

        def step(h, buf, gsem, osem, t):
            drain_g(buf, gsem)

            @pl.when(t > 0)
            def _():
                drain_o(buf, osem)

            transpose(buf)
            for q in range(NBT):
                pltpu.async_copy(
                    T.at[buf, q, :, :, pl.ds(0, 128)],
                    o5.at[h, :, wid * NBT + q, :, :],
                    osem,
                )

            @pl.when(h + 2 < HIST)
            def _():
                fire(h + 2, buf, gsem)

        fire(0, 0, gs0)
        fire(1, 1, gs1)

        def pair(t, carry):
            step(2 * t, 0, gs0, os0, t)
            step(2 * t + 1, 1, gs1, os1, t)
            return carry

        lax.fori_loop(0, HIST // 2, pair, 0)
        drain_o(0, os0)
        drain_o(1, os1)

    return gather_kernel


_repack = _make_repack()
_gather = _make_gather()


@jax.jit
def kernel(x, weight):
    xv = x.T.reshape(HIST, BATCH // 128, 128).astype(jnp.int32)
    tail = lax.slice(weight, (NFULL * CH, 0), (NUM_ROWS, DIM)).reshape(-1)
    wlin = _repack(weight.T, tail)
    o5 = _gather(xv, wlin.reshape(NUM_ROWS, DIM))
    out = (
        o5.transpose(0, 1, 3, 2, 4)
        .reshape(HIST, DIM, BATCH)
        .transpose(2, 0, 1)
    )
    return out


# speedup vs baseline: 2.1674x; 2.0399x over previous
import functools

import jax
import jax.numpy as jnp
from jax import lax
from jax.experimental import pallas as pl
from jax.experimental.pallas import tpu as pltpu
from jax.experimental.pallas import tpu_sc as plsc

NUM_ROWS = 1000000
DIM = 32
BATCH = 16384
HIST = 50

_info = plsc.get_sparse_core_info()
NC = _info.num_cores
NS = _info.num_subcores
NW = NC * NS

BLK = BATCH // NW
NBT = BLK // 128
NDT = DIM // 8

CH = 128
NFULL = NUM_ROWS // CH
TAIL = NUM_ROWS - NFULL * CH
NKP = (-(-NFULL // NW) + 1) // 2

_mesh = plsc.VectorSubcoreMesh(core_axis_name="c", subcore_axis_name="s")


def _make_repack():
    @functools.partial(
        pl.kernel,
        mesh=_mesh,
        out_type=jax.ShapeDtypeStruct((NUM_ROWS * DIM,), jnp.float32),
        scratch_types=[
            pltpu.VMEM((32, CH + 8), jnp.float32),
            pltpu.VMEM((32, CH + 8), jnp.float32),
            pltpu.VMEM((CH * DIM,), jnp.float32),
            pltpu.VMEM((CH * DIM,), jnp.float32),
            pltpu.VMEM((CH * 33,), jnp.float32),
            pltpu.VMEM((CH * 33,), jnp.float32),
            pltpu.SemaphoreType.DMA,
            pltpu.SemaphoreType.DMA,
            pltpu.SemaphoreType.DMA,
            pltpu.SemaphoreType.DMA,
        ],
        compiler_params=pltpu.CompilerParams(
            use_tc_tiling_on_sc=True, needs_layout_passes=False
        ),
    )
    def repack_kernel(wt_hbm, tail_hbm, wlin, Sa, Sb, Ta, Tb, Pa, Pb,
                      is0, is1, os0, os1):
        wid = lax.axis_index("s") * NC + lax.axis_index("c")

        iota16 = lax.iota(jnp.int32, 16)
        iota33 = iota16 * 33

        def chunk_of(k):
            return wid + k * NW

        def fire_in(k, S, isem):
            @pl.when(chunk_of(k) < NFULL)
            def _():
                pltpu.async_copy(
                    wt_hbm.at[:, pl.ds(chunk_of(k) * CH, CH)],
                    S.at[:, pl.ds(0, CH)],
                    isem,
                )

        def drain_in(S, isem):
            pltpu.make_async_copy(
                wt_hbm.at[:, pl.ds(0, CH)], S.at[:, pl.ds(0, CH)], isem
            ).wait()

        def drain_out(T, osem):
            pltpu.make_async_copy(T, wlin.at[pl.ds(0, CH * DIM)], osem).wait()

        def work(k, S, T, P, isem, osem, t):
            @pl.when(chunk_of(k) < NFULL)
            def _():
                drain_in(S, isem)

                @pl.when(t > 0)
                def _():
                    drain_out(T, osem)

                @plsc.parallel_loop(0, 2 * CH, 1, unroll=8)
                def _body(j):
                    d = lax.shift_right_logical(j, 3)
                    bq = lax.bitwise_and(j, 7)
                    v = S[d, pl.ds(bq * 16, 16)]
                    plsc.store_scatter(P, [iota33 + (bq * 528 + d)], v)

                @plsc.parallel_loop(0, 2 * CH, 1, unroll=8)
                def _body2(j):
                    bc = lax.shift_right_logical(j, 1)
                    d0 = lax.bitwise_and(j, 1) * 16
                    v = P[pl.ds(bc * 33 + d0, 16)]
                    T[pl.ds(bc * 32 + d0, 16)] = v

                pltpu.async_copy(
                    T, wlin.at[pl.ds(chunk_of(k) * CH * DIM, CH * DIM)], osem
                )
                fire_in(k + 2, S, isem)

        fire_in(0, Sa, is0)
        fire_in(1, Sb, is1)

        def pair(t, carry):
            work(2 * t, Sa, Ta, Pa, is0, os0, t)
            work(2 * t + 1, Sb, Tb, Pb, is1, os1, t)
            return carry

        lax.fori_loop(0, NKP, pair, 0)

        drain_out(Ta, os0)
        drain_out(Tb, os1)

        @pl.when(wid == 0)
        def _():
            pltpu.sync_copy(tail_hbm, Ta.at[pl.ds(0, TAIL * DIM)])
            pltpu.sync_copy(
                Ta.at[pl.ds(0, TAIL * DIM)],
                wlin.at[pl.ds(NFULL * CH * DIM, TAIL * DIM)],
            )

    return repack_kernel


def _make_gather():
    @functools.partial(
        pl.kernel,
        mesh=_mesh,
        out_type=jax.ShapeDtypeStruct((HIST, NDT, BATCH // 128, 8, 128),
                                      jnp.float32),
        scratch_types=[
            pltpu.VMEM((HIST, NBT, 128), jnp.int32),
            pltpu.VMEM((2, BLK, DIM), jnp.float32),
            pltpu.VMEM((2, NBT, NDT, 8, 131), jnp.float32),
            pltpu.SemaphoreType.DMA,
            pltpu.SemaphoreType.DMA,
            pltpu.SemaphoreType.DMA,
            pltpu.SemaphoreType.DMA,
        ],
        compiler_params=pltpu.CompilerParams(
            use_tc_tiling_on_sc=False, needs_layout_passes=False
        ),
    )
    def gather_kernel(x_hbm, w_hbm, o5, idx_v, G, T, gs0, gs1, os0, os1):
        wid = lax.axis_index("s") * NC + lax.axis_index("c")
        pltpu.sync_copy(x_hbm.at[:, pl.ds(wid * NBT, NBT), :], idx_v)

        def fire(h, buf, gsem):
            for q in range(NBT):
                pltpu.async_copy(
                    w_hbm.at[idx_v.at[h, q]],
                    G.at[buf, pl.ds(q * 128, 128)],
                    gsem,
                )

        def drain_g(buf, gsem):
            pltpu.make_async_copy(
                w_hbm.at[pl.ds(0, BLK)], G.at[buf], gsem
            ).wait()

        def drain_o(buf, osem):
            pltpu.make_async_copy(
                T.at[buf, :, :, :, pl.ds(0, 128)],
                o5.at[0, :, pl.ds(wid * NBT, NBT), :, :],
                osem,
            ).wait()

        iota16 = lax.iota(jnp.int32, 16)
        iota33 = iota16 * 33
        dt_lo = lax.shift_right_logical(iota16, 3)
        dt_hi = dt_lo + 2
        dr_v = lax.bitwise_and(iota16, 7)

        def transpose(buf):
            @plsc.parallel_loop(0, BLK, 1, unroll=8)
            def _body(r):
                q = lax.shift_right_logical(r, 7)
                bc = lax.bitwise_and(r, 127)
                qv = jnp.full((16,), q, jnp.int32)
                bcv = jnp.full((16,), bc, jnp.int32)
                v0 = G[buf, r, pl.ds(0, 16)]
                v1 = G[buf, r, pl.ds(16, 16)]
                plsc.store_scatter(T.at[buf], [qv, dt_lo, dr_v, bcv], v0)
                plsc.store_scatter(T.at[buf], [qv, dt_hi, dr_v, bcv], v1)

        def step(h, buf, gsem, osem, t):
            drain_g(buf, gsem)

            @pl.when(t > 0)
            def _():
                drain_o(buf, osem)

            transpose(buf)
            for q in range(NBT):
                pltpu.async_copy(
                    T.at[buf, q, :, :, pl.ds(0, 128)],
                    o5.at[h, :, wid * NBT + q, :, :],
                    osem,
                )

            @pl.when(h + 2 < HIST)
            def _():
                fire(h + 2, buf, gsem)

        fire(0, 0, gs0)
        fire(1, 1, gs1)

        def pair(t, carry):
            step(2 * t, 0, gs0, os0, t)
            step(2 * t + 1, 1, gs1, os1, t)
            return carry

        lax.fori_loop(0, HIST // 2, pair, 0)
        drain_o(0, os0)
        drain_o(1, os1)

    return gather_kernel


_repack = _make_repack()
_gather = _make_gather()


@jax.jit
def kernel(x, weight):
    xv = x.T.reshape(HIST, BATCH // 128, 128).astype(jnp.int32)
    tail = lax.slice(weight, (NFULL * CH, 0), (NUM_ROWS, DIM)).reshape(-1)
    wlin = _repack(weight.T, tail)
    o5 = _gather(xv, wlin.reshape(NUM_ROWS, DIM))
    out = (
        o5.transpose(0, 1, 3, 2, 4)
        .reshape(HIST, DIM, BATCH)
        .transpose(2, 0, 1)
    )
    return out
